# initial kernel scaffold (unmeasured)
import jax
import jax.numpy as jnp
from jax import lax
from jax.experimental import pallas as pl
from jax.experimental.pallas import tpu as pltpu

C = 128
MAXC = 32


def kernel(x, dest):
    T, D = x.shape
    my_x = lax.axis_index("x")

    keep = (dest == my_x).astype(jnp.int32)
    n_keep = jnp.sum(keep)
    n_send = T - n_keep

    perm = jnp.argsort(keep, stable=True)
    send_buf = jnp.take(x, perm, axis=0)

    n_chunks = (n_send + C - 1) // C

    def body(nc_ref, send_ref, out_ref, send_sems, recv_sems):
        mx = lax.axis_index("x")
        my = lax.axis_index("y")
        mz = lax.axis_index("z")
        peer = (1 - mx, my, mz)

        barrier = pltpu.get_barrier_semaphore()
        pl.semaphore_signal(
            barrier, inc=1, device_id=peer, device_id_type=pl.DeviceIdType.MESH
        )
        pl.semaphore_wait(barrier, 1)

        nc = nc_ref[0]
        descs = []
        for i in range(MAXC):
            rdma = pltpu.make_async_remote_copy(
                src_ref=send_ref.at[pl.ds(i * C, C)],
                dst_ref=out_ref.at[pl.ds(i * C, C)],
                send_sem=send_sems.at[i],
                recv_sem=recv_sems.at[i],
                device_id=peer,
                device_id_type=pl.DeviceIdType.MESH,
            )
            descs.append(rdma)

            @pl.when(i < nc)
            def _(rdma=rdma):
                rdma.start()

        for i in range(MAXC):

            @pl.when(i < nc)
            def _(r=descs[i]):
                r.wait_send()
                r.wait_recv()

    recv = pl.pallas_call(
        body,
        out_shape=jax.ShapeDtypeStruct((T, D), x.dtype),
        in_specs=[
            pl.BlockSpec(memory_space=pltpu.SMEM),
            pl.BlockSpec(memory_space=pltpu.ANY),
        ],
        out_specs=pl.BlockSpec(memory_space=pltpu.ANY),
        scratch_shapes=[
            pltpu.SemaphoreType.DMA((MAXC,)),
            pltpu.SemaphoreType.DMA((MAXC,)),
        ],
        compiler_params=pltpu.CompilerParams(collective_id=0),
    )(jnp.reshape(n_chunks, (1,)).astype(jnp.int32), send_buf)

    j = jnp.arange(T)
    merged = jnp.where((j < n_send)[:, None], recv, send_buf)
    shift = jnp.where(my_x == 0, n_send, 0)
    return jnp.take(merged, (j + shift) % T, axis=0)


# baseline (device time: 672341 ns/iter reference)
import jax
import jax.numpy as jnp
from jax import lax
from jax.experimental import pallas as pl
from jax.experimental.pallas import tpu as pltpu

C = 128
MAXC = 32


def kernel(x, dest):
    T, D = x.shape
    my_x = lax.axis_index("x")

    keep = (dest == my_x).astype(jnp.int32)
    n_keep = jnp.sum(keep)
    n_send = T - n_keep

    perm = jnp.argsort(keep, stable=True)
    send_buf = jnp.take(x, perm, axis=0)

    n_chunks = (n_send + C - 1) // C

    def body(nc_ref, send_ref, out_ref, send_sems, recv_sems):
        mx = lax.axis_index("x")
        my = lax.axis_index("y")
        mz = lax.axis_index("z")
        peer = (1 - mx, my, mz)

        barrier = pltpu.get_barrier_semaphore()
        pl.semaphore_signal(
            barrier, inc=1, device_id=peer, device_id_type=pl.DeviceIdType.MESH
        )
        pl.semaphore_wait(barrier, 1)

        nc = nc_ref[0]
        descs = []
        for i in range(MAXC):
            rdma = pltpu.make_async_remote_copy(
                src_ref=send_ref.at[pl.ds(i * C, C)],
                dst_ref=out_ref.at[pl.ds(i * C, C)],
                send_sem=send_sems.at[i],
                recv_sem=recv_sems.at[i],
                device_id=peer,
                device_id_type=pl.DeviceIdType.MESH,
            )
            descs.append(rdma)

            @pl.when(i < nc)
            def _(rdma=rdma):
                rdma.start()

        for i in range(MAXC):

            @pl.when(i < nc)
            def _(r=descs[i]):
                r.wait_send()
                r.wait_recv()

    recv = pl.pallas_call(
        body,
        out_shape=jax.ShapeDtypeStruct((T, D), x.dtype),
        in_specs=[
            pl.BlockSpec(memory_space=pltpu.SMEM),
            pl.BlockSpec(memory_space=pltpu.MemorySpace.HBM),
        ],
        out_specs=pl.BlockSpec(memory_space=pltpu.MemorySpace.HBM),
        scratch_shapes=[
            pltpu.SemaphoreType.DMA((MAXC,)),
            pltpu.SemaphoreType.DMA((MAXC,)),
        ],
        compiler_params=pltpu.CompilerParams(collective_id=0),
    )(jnp.reshape(n_chunks, (1,)).astype(jnp.int32), send_buf)

    j = jnp.arange(T)
    merged = jnp.where((j < n_send)[:, None], recv, send_buf)
    shift = jnp.where(my_x == 0, n_send, 0)
    return jnp.take(merged, (j + shift) % T, axis=0)


# device time: 162322 ns/iter; 4.1420x vs baseline; 4.1420x over previous
import jax
import jax.numpy as jnp
from jax import lax
from jax.experimental import pallas as pl
from jax.experimental.pallas import tpu as pltpu


def kernel(x, dest):
    T, D = x.shape
    my_x = lax.axis_index("x")

    x3 = jnp.reshape(x, (T, 8, 128))

    keep = dest == my_x
    keep_i = keep.astype(jnp.int32)
    n_keep = jnp.sum(keep_i)
    n_send = T - n_keep

    kpos = jnp.cumsum(keep_i) - 1
    spos = jnp.cumsum(1 - keep_i) - 1
    keep_base = jnp.where(my_x == 0, 0, n_send)
    send_base = jnp.where(my_x == 1, n_keep, 0)
    tgt = jnp.where(keep, keep_base + kpos, send_base + spos)
    enc = jnp.where(keep, tgt, -tgt - 1).astype(jnp.int32)
    scalars = jnp.stack([n_keep, n_send]).astype(jnp.int32)

    def body(scal_ref, enc_ref, x_ref, out_ref, local_sem, send_sem, recv_sem):
        mx = lax.axis_index("x")
        my_ = lax.axis_index("y")
        mz = lax.axis_index("z")
        peer = (1 - mx, my_, mz)

        barrier = pltpu.get_barrier_semaphore()
        pl.semaphore_signal(
            barrier, inc=1, device_id=peer, device_id_type=pl.DeviceIdType.MESH
        )
        pl.semaphore_wait(barrier, 1)

        nk = scal_ref[0]
        ns = scal_ref[1]

        def issue(i, c):
            e = enc_ref[i]
            is_keep = e >= 0
            t = jnp.where(is_keep, e, -e - 1)
            src = x_ref.at[pl.ds(i, 1)]
            dst = out_ref.at[pl.ds(t, 1)]

            @pl.when(is_keep)
            def _():
                pltpu.make_async_copy(src, dst, local_sem).start()

            @pl.when(jnp.logical_not(is_keep))
            def _():
                pltpu.make_async_remote_copy(
                    src_ref=src,
                    dst_ref=dst,
                    send_sem=send_sem,
                    recv_sem=recv_sem,
                    device_id=peer,
                    device_id_type=pl.DeviceIdType.MESH,
                ).start()

            return c

        lax.fori_loop(0, T, issue, 0)

        dummy_src = x_ref.at[pl.ds(0, 1)]
        dummy_dst = out_ref.at[pl.ds(0, 1)]

        def mk_remote():
            return pltpu.make_async_remote_copy(
                src_ref=dummy_src,
                dst_ref=dummy_dst,
                send_sem=send_sem,
                recv_sem=recv_sem,
                device_id=peer,
                device_id_type=pl.DeviceIdType.MESH,
            )

        def wait_local(i, c):
            pltpu.make_async_copy(dummy_src, dummy_dst, local_sem).wait()
            return c

        lax.fori_loop(0, nk, wait_local, 0)

        def wait_send(i, c):
            mk_remote().wait_send()
            return c

        lax.fori_loop(0, ns, wait_send, 0)

        def wait_recv(i, c):
            mk_remote().wait_recv()
            return c

        lax.fori_loop(0, ns, wait_recv, 0)

    out3 = pl.pallas_call(
        body,
        out_shape=jax.ShapeDtypeStruct((T, 8, 128), x.dtype),
        in_specs=[
            pl.BlockSpec(memory_space=pltpu.SMEM),
            pl.BlockSpec(memory_space=pltpu.SMEM),
            pl.BlockSpec(memory_space=pltpu.MemorySpace.HBM),
        ],
        out_specs=pl.BlockSpec(memory_space=pltpu.MemorySpace.HBM),
        scratch_shapes=[
            pltpu.SemaphoreType.DMA,
            pltpu.SemaphoreType.DMA,
            pltpu.SemaphoreType.DMA,
        ],
        compiler_params=pltpu.CompilerParams(collective_id=0),
    )(scalars, enc, x3)
    return jnp.reshape(out3, (T, D))


# device time: 146862 ns/iter; 4.5780x vs baseline; 1.1053x over previous
import jax
import jax.numpy as jnp
from jax import lax
from jax.experimental import pallas as pl
from jax.experimental.pallas import tpu as pltpu

W = 32


def kernel(x, dest):
    T, D = x.shape
    my_x = lax.axis_index("x")

    x3 = jnp.reshape(x, (T, 8, 128))

    keep = dest == my_x
    n_keep = jnp.sum(keep.astype(jnp.int32))
    n_send = T - n_keep

    send_src = jnp.nonzero(~keep, size=T, fill_value=0)[0].astype(jnp.int32)
    keep_src = jnp.nonzero(keep, size=T, fill_value=0)[0].astype(jnp.int32)

    keep_base = jnp.where(my_x == 0, 0, n_send)
    send_base = jnp.where(my_x == 1, n_keep, 0)
    scalars = jnp.stack([n_keep, n_send, keep_base, send_base]).astype(jnp.int32)

    def body(scal_ref, ssrc_ref, ksrc_ref, x_ref, out_ref, local_sem, send_sem, recv_sem):
        mx = lax.axis_index("x")
        my_ = lax.axis_index("y")
        mz = lax.axis_index("z")
        peer = (1 - mx, my_, mz)

        barrier = pltpu.get_barrier_semaphore()
        pl.semaphore_signal(
            barrier, inc=1, device_id=peer, device_id_type=pl.DeviceIdType.MESH
        )
        pl.semaphore_wait(barrier, 1)

        nk = scal_ref[0]
        ns = scal_ref[1]
        kb = scal_ref[2]
        sb = scal_ref[3]

        def send_it(i, c):
            s = ssrc_ref[i]
            pltpu.make_async_remote_copy(
                src_ref=x_ref.at[pl.ds(s, 1)],
                dst_ref=out_ref.at[pl.ds(sb + i, 1)],
                send_sem=send_sem,
                recv_sem=recv_sem,
                device_id=peer,
                device_id_type=pl.DeviceIdType.MESH,
            ).start()
            return c

        lax.fori_loop(0, ns, send_it, 0)

        def keep_it(i, c):
            s = ksrc_ref[i]
            pltpu.make_async_copy(
                x_ref.at[pl.ds(s, 1)], out_ref.at[pl.ds(kb + i, 1)], local_sem
            ).start()
            return c

        lax.fori_loop(0, nk, keep_it, 0)

        def mk_remote(rows):
            return pltpu.make_async_remote_copy(
                src_ref=x_ref.at[pl.ds(0, rows)],
                dst_ref=out_ref.at[pl.ds(0, rows)],
                send_sem=send_sem,
                recv_sem=recv_sem,
                device_id=peer,
                device_id_type=pl.DeviceIdType.MESH,
            )

        def mk_local(rows):
            return pltpu.make_async_copy(
                x_ref.at[pl.ds(0, rows)], out_ref.at[pl.ds(0, rows)], local_sem
            )

        def drain(n, wait_batch, wait_one):
            def wb(i, c):
                wait_batch()
                return c

            lax.fori_loop(0, n // W, wb, 0)

            def w1(i, c):
                wait_one()
                return c

            lax.fori_loop(0, n % W, w1, 0)

        drain(nk, lambda: mk_local(W).wait(), lambda: mk_local(1).wait())
        drain(ns, lambda: mk_remote(W).wait_send(), lambda: mk_remote(1).wait_send())
        drain(ns, lambda: mk_remote(W).wait_recv(), lambda: mk_remote(1).wait_recv())

    out3 = pl.pallas_call(
        body,
        out_shape=jax.ShapeDtypeStruct((T, 8, 128), x.dtype),
        in_specs=[
            pl.BlockSpec(memory_space=pltpu.SMEM),
            pl.BlockSpec(memory_space=pltpu.SMEM),
            pl.BlockSpec(memory_space=pltpu.SMEM),
            pl.BlockSpec(memory_space=pltpu.MemorySpace.HBM),
        ],
        out_specs=pl.BlockSpec(memory_space=pltpu.MemorySpace.HBM),
        scratch_shapes=[
            pltpu.SemaphoreType.DMA,
            pltpu.SemaphoreType.DMA,
            pltpu.SemaphoreType.DMA,
        ],
        compiler_params=pltpu.CompilerParams(collective_id=0),
    )(scalars, send_src, keep_src, x3)
    return jnp.reshape(out3, (T, D))


# device time: 134320 ns/iter; 5.0055x vs baseline; 1.0934x over previous
import jax
import jax.numpy as jnp
from jax import lax
from jax.experimental import pallas as pl
from jax.experimental.pallas import tpu as pltpu

W = 128


def kernel(x, dest):
    T, D = x.shape
    my_x = lax.axis_index("x")

    x3 = jnp.reshape(x, (T, 8, 128))

    n_keep = jnp.sum((dest == my_x).astype(jnp.int32))
    n_send = T - n_keep

    keep_base = jnp.where(my_x == 0, 0, n_send)
    send_base = jnp.where(my_x == 1, n_keep, 0)
    scalars = jnp.stack([n_keep, n_send, keep_base, send_base]).astype(jnp.int32)

    def body(scal_ref, dest_ref, x_ref, out_ref, local_sem, send_sem, recv_sem):
        mx = lax.axis_index("x")
        my_ = lax.axis_index("y")
        mz = lax.axis_index("z")
        peer = (1 - mx, my_, mz)

        barrier = pltpu.get_barrier_semaphore()
        pl.semaphore_signal(
            barrier, inc=1, device_id=peer, device_id_type=pl.DeviceIdType.MESH
        )
        pl.semaphore_wait(barrier, 1)

        nk = scal_ref[0]
        ns = scal_ref[1]
        kb = scal_ref[2]
        sb = scal_ref[3]

        def it(i, carry):
            cs, ck = carry
            is_keep = dest_ref[i] == mx
            src = x_ref.at[pl.ds(i, 1)]

            @pl.when(is_keep)
            def _():
                pltpu.make_async_copy(
                    src, out_ref.at[pl.ds(kb + ck, 1)], local_sem
                ).start()

            @pl.when(jnp.logical_not(is_keep))
            def _():
                pltpu.make_async_remote_copy(
                    src_ref=src,
                    dst_ref=out_ref.at[pl.ds(sb + cs, 1)],
                    send_sem=send_sem,
                    recv_sem=recv_sem,
                    device_id=peer,
                    device_id_type=pl.DeviceIdType.MESH,
                ).start()

            ik = is_keep.astype(jnp.int32)
            return (cs + 1 - ik, ck + ik)

        lax.fori_loop(0, T, it, (jnp.int32(0), jnp.int32(0)))

        def mk_remote(rows):
            return pltpu.make_async_remote_copy(
                src_ref=x_ref.at[pl.ds(0, rows)],
                dst_ref=out_ref.at[pl.ds(0, rows)],
                send_sem=send_sem,
                recv_sem=recv_sem,
                device_id=peer,
                device_id_type=pl.DeviceIdType.MESH,
            )

        def mk_local(rows):
            return pltpu.make_async_copy(
                x_ref.at[pl.ds(0, rows)], out_ref.at[pl.ds(0, rows)], local_sem
            )

        def drain(n, wait_batch, wait_one):
            def wb(i, c):
                wait_batch()
                return c

            lax.fori_loop(0, n // W, wb, 0)

            def w1(i, c):
                wait_one()
                return c

            lax.fori_loop(0, n % W, w1, 0)

        drain(nk, lambda: mk_local(W).wait(), lambda: mk_local(1).wait())
        drain(ns, lambda: mk_remote(W).wait_send(), lambda: mk_remote(1).wait_send())
        drain(ns, lambda: mk_remote(W).wait_recv(), lambda: mk_remote(1).wait_recv())

    out3 = pl.pallas_call(
        body,
        out_shape=jax.ShapeDtypeStruct((T, 8, 128), x.dtype),
        in_specs=[
            pl.BlockSpec(memory_space=pltpu.SMEM),
            pl.BlockSpec(memory_space=pltpu.SMEM),
            pl.BlockSpec(memory_space=pltpu.MemorySpace.HBM),
        ],
        out_specs=pl.BlockSpec(memory_space=pltpu.MemorySpace.HBM),
        scratch_shapes=[
            pltpu.SemaphoreType.DMA,
            pltpu.SemaphoreType.DMA,
            pltpu.SemaphoreType.DMA,
        ],
        compiler_params=pltpu.CompilerParams(collective_id=0),
    )(scalars, dest.astype(jnp.int32), x3)
    return jnp.reshape(out3, (T, D))
